# Initial kernel scaffold; baseline (speedup 1.0000x reference)
import jax
import jax.numpy as jnp
from jax.experimental import pallas as pl


def kernel(lang, text, seq_len, table):
    raise NotImplementedError("write your pallas kernel here")


# speedup vs baseline: 5.6269x; 5.6269x over previous
import functools

import jax
import jax.numpy as jnp
from jax import lax
from jax.experimental import pallas as pl
from jax.experimental.pallas import tpu as pltpu
from jax.experimental.pallas import tpu_sc as plsc

NC = 2
NS = 16
L = 16
NW = NC * NS

B = 1024
S = 200
D = 128
TOTAL = B * S
PER_W = TOTAL // NW
CHUNK = 128
NCHUNK = PER_W // CHUNK
VPC = CHUNK // L


def _sc_gather(table, text_rows, seql):
    mesh = plsc.VectorSubcoreMesh(core_axis_name="c", subcore_axis_name="s")

    @functools.partial(
        pl.kernel,
        out_type=jax.ShapeDtypeStruct((TOTAL, D), jnp.float32),
        mesh=mesh,
        scratch_types=[
            pltpu.VMEM((NCHUNK, CHUNK), jnp.int32),
            pltpu.VMEM((16,), jnp.int32),
            pltpu.VMEM((CHUNK, D), jnp.float32),
            pltpu.SemaphoreType.DMA,
        ],
    )
    def k(table_hbm, text_hbm, seql_hbm, out_hbm, idx_v, seql_v, rows_v, sem):
        wid = lax.axis_index("s") * NC + lax.axis_index("c")
        base = wid * PER_W
        pltpu.sync_copy(text_hbm.at[wid], idx_v)
        pltpu.sync_copy(seql_hbm, seql_v)
        seql = seql_v[...]

        @pl.loop(0, NCHUNK)
        def _fix(r):
            for v in range(VPC):
                off = r * CHUNK + v * L
                vec = idx_v[r, pl.ds(v * L, L)]
                pos = lax.rem(off + lax.iota(jnp.int32, L), S)
                idx_v[r, pl.ds(v * L, L)] = jnp.where(pos < seql, vec + 1, 0)

        @pl.loop(0, NCHUNK)
        def _gather(c):
            pltpu.async_copy(table_hbm.at[idx_v.at[c]], rows_v, sem).wait()
            pltpu.sync_copy(rows_v, out_hbm.at[pl.ds(base + c * CHUNK, CHUNK)])

    return k(table, text_rows, seql)


def kernel(lang, text, seq_len, table):
    del lang
    text_rows = text.astype(jnp.int32).reshape(NW, NCHUNK, CHUNK)
    seql = jnp.full((16,), seq_len, dtype=jnp.int32)
    out = _sc_gather(table, text_rows, seql)
    return out.reshape(B, S, D)


# speedup vs baseline: 6.5250x; 1.1596x over previous
import functools

import jax
import jax.numpy as jnp
from jax import lax
from jax.experimental import pallas as pl
from jax.experimental.pallas import tpu as pltpu
from jax.experimental.pallas import tpu_sc as plsc

NC = 2
NS = 16
L = 16
NW = NC * NS

B = 1024
S = 200
D = 128
TOTAL = B * S
PER_W = TOTAL // NW
CHUNK = 128
NCHUNK = PER_W // CHUNK
VPC = CHUNK // L


def _sc_gather(table, text_rows, seql):
    mesh = plsc.VectorSubcoreMesh(core_axis_name="c", subcore_axis_name="s")

    @functools.partial(
        pl.kernel,
        out_type=jax.ShapeDtypeStruct((TOTAL, D), jnp.float32),
        mesh=mesh,
        scratch_types=[
            pltpu.VMEM((NCHUNK, CHUNK), jnp.int32),
            pltpu.VMEM((16,), jnp.int32),
            pltpu.VMEM((CHUNK, D), jnp.float32),
            pltpu.VMEM((CHUNK, D), jnp.float32),
            pltpu.SemaphoreType.DMA,
            pltpu.SemaphoreType.DMA,
            pltpu.SemaphoreType.DMA,
            pltpu.SemaphoreType.DMA,
        ],
    )
    def k(table_hbm, text_hbm, seql_hbm, out_hbm, idx_v, seql_v,
          buf0, buf1, gsem0, gsem1, ssem0, ssem1):
        wid = lax.axis_index("s") * NC + lax.axis_index("c")
        base = wid * PER_W
        bufs = (buf0, buf1)
        gsems = (gsem0, gsem1)
        ssems = (ssem0, ssem1)
        pltpu.sync_copy(text_hbm.at[wid], idx_v)
        pltpu.sync_copy(seql_hbm, seql_v)
        seql = seql_v[...]

        @pl.loop(0, NCHUNK)
        def _fix(r):
            for v in range(VPC):
                off = r * CHUNK + v * L
                vec = idx_v[r, pl.ds(v * L, L)]
                pos = lax.rem(off + lax.iota(jnp.int32, L), S)
                idx_v[r, pl.ds(v * L, L)] = jnp.where(pos < seql, vec + 1, 0)

        def start_gather(c, b):
            pltpu.async_copy(table_hbm.at[idx_v.at[c]], bufs[b], gsems[b])

        def wait_gather(b):
            pltpu.make_async_copy(table_hbm.at[idx_v.at[0]], bufs[b],
                                  gsems[b]).wait()

        def start_store(c, b):
            pltpu.async_copy(bufs[b], out_hbm.at[pl.ds(base + c * CHUNK, CHUNK)],
                             ssems[b])

        def wait_store(b):
            pltpu.make_async_copy(bufs[b], out_hbm.at[pl.ds(base, CHUNK)],
                                  ssems[b]).wait()

        start_gather(0, 0)

        @pl.loop(0, NCHUNK, step=2)
        def _pipe(c):
            for b in range(2):
                chunk = c + b
                wait_gather(b)
                if b == 0:
                    @pl.when(c > 0)
                    def _():
                        wait_store(1)
                    start_gather(chunk + 1, 1)
                else:
                    wait_store(0)

                    @pl.when(c < NCHUNK - 2)
                    def _():
                        start_gather(chunk + 1, 0)
                start_store(chunk, b)

        wait_store(1)

    return k(table, text_rows, seql)


def kernel(lang, text, seq_len, table):
    del lang
    text_rows = text.astype(jnp.int32).reshape(NW, NCHUNK, CHUNK)
    seql = jnp.full((16,), seq_len, dtype=jnp.int32)
    out = _sc_gather(table, text_rows, seql)
    return out.reshape(B, S, D)


# speedup vs baseline: 7.8049x; 1.1962x over previous
import functools

import jax
import jax.numpy as jnp
from jax import lax
from jax.experimental import pallas as pl
from jax.experimental.pallas import tpu as pltpu
from jax.experimental.pallas import tpu_sc as plsc

NC = 2
NS = 16
L = 16
NW = NC * NS

B = 1024
S = 200
D = 128
TOTAL = B * S
PER_W = TOTAL // NW
CHUNK = 128
NCHUNK = PER_W // CHUNK
VPC = CHUNK // L
DEPTH = 5


def _sc_gather(table, text_rows, seql):
    mesh = plsc.VectorSubcoreMesh(core_axis_name="c", subcore_axis_name="s")

    @functools.partial(
        pl.kernel,
        out_type=jax.ShapeDtypeStruct((TOTAL, D), jnp.float32),
        mesh=mesh,
        scratch_types=[
            pltpu.VMEM((NCHUNK, CHUNK), jnp.int32),
            pltpu.VMEM((16,), jnp.int32),
            [pltpu.VMEM((CHUNK, D), jnp.float32) for _ in range(DEPTH)],
            [pltpu.SemaphoreType.DMA for _ in range(DEPTH)],
            [pltpu.SemaphoreType.DMA for _ in range(DEPTH)],
        ],
    )
    def k(table_hbm, text_hbm, seql_hbm, out_hbm, idx_v, seql_v,
          bufs, gsems, ssems):
        wid = lax.axis_index("s") * NC + lax.axis_index("c")
        base = wid * PER_W
        pltpu.sync_copy(text_hbm.at[wid], idx_v)
        pltpu.sync_copy(seql_hbm, seql_v)
        seql = seql_v[...]
        lane = lax.iota(jnp.int32, L)

        def fix(r):
            for v in range(VPC):
                off = r * CHUNK + v * L
                vec = idx_v[r, pl.ds(v * L, L)]
                pos = lax.rem(off + lane, S)
                idx_v[r, pl.ds(v * L, L)] = jnp.where(pos < seql, vec + 1, 0)

        def start_gather(c, b):
            pltpu.async_copy(table_hbm.at[idx_v.at[c]], bufs[b], gsems[b])

        def wait_gather(b):
            pltpu.make_async_copy(table_hbm.at[idx_v.at[0]], bufs[b],
                                  gsems[b]).wait()

        def start_store(c, b):
            pltpu.async_copy(bufs[b], out_hbm.at[pl.ds(base + c * CHUNK, CHUNK)],
                             ssems[b])

        def wait_store(b):
            pltpu.make_async_copy(bufs[b], out_hbm.at[pl.ds(base, CHUNK)],
                                  ssems[b]).wait()

        for j in range(DEPTH - 1):
            fix(j)
            start_gather(j, j)

        @pl.loop(0, NCHUNK, step=DEPTH)
        def _pipe(cbase):
            for b in range(DEPTH):
                c = cbase + b
                wait_gather(b)
                start_store(c, b)
                nxt = c + DEPTH - 1
                bn = (b + DEPTH - 1) % DEPTH

                @pl.when(nxt < NCHUNK)
                def _():
                    fix(nxt)

                    @pl.when(c >= 1)
                    def _():
                        wait_store(bn)
                    start_gather(nxt, bn)

        for j in range(DEPTH):
            wait_store((NCHUNK - DEPTH + j) % DEPTH)

    return k(table, text_rows, seql)


def kernel(lang, text, seq_len, table):
    del lang
    text_rows = text.astype(jnp.int32).reshape(NW, NCHUNK, CHUNK)
    seql = jnp.full((16,), seq_len, dtype=jnp.int32)
    out = _sc_gather(table, text_rows, seql)
    return out.reshape(B, S, D)
